# baseline (speedup 1.0000x reference)
import jax
import jax.numpy as jnp
from jax.experimental import pallas as pl
from jax.experimental.pallas import tpu as pltpu

_N = 128
_C = 64
_S = 25
_K = 26
_EPS_DIV = 2.220446049250313e-16
_BN_S = 1.0 / (1.0 + 1e-5) ** 0.5

_R_XS = 0
_R_W1 = 3200
_R_W2 = 3776
_R_FC3W = 3792
_R_G1 = 3800
_R_B1C = 3801
_R_B1B = 3802
_R_FC3B = 3803
_R_FC4W = 3804
_ROWS = 3808


def _body(big_ref, feat_ref, gcnw_ref, p2_ref, scal_ref, out_ref):
    f32 = jnp.float32
    sc2 = scal_ref[0]
    cb2 = scal_ref[1]
    cb4 = scal_ref[2]
    a0 = scal_ref[3]
    a1 = scal_ref[4]
    a2 = scal_ref[5]

    xs = big_ref[_R_XS:_R_XS + _S * _N, :]
    srow = jax.lax.broadcasted_iota(jnp.int32, (_S * _N, _C), 0) // _N
    si = srow // 5
    sj = srow - 5 * si
    y1 = None
    for di in range(3):
        for dj in range(3):
            off = di * 3 + dj
            dshift = (di - 1) * 5 + (dj - 1)
            xsh = jnp.roll(xs, -dshift * _N, axis=0) if dshift else xs
            ii = si + (di - 1)
            jj = sj + (dj - 1)
            valid = (ii >= 0) & (ii < 5) & (jj >= 0) & (jj < 5)
            xm = jnp.where(valid, xsh, 0.0)
            w = big_ref[_R_W1 + off * _C:_R_W1 + (off + 1) * _C, :]
            t = jax.lax.dot_general(xm, w, (((1,), (0,)), ((), ())),
                                    preferred_element_type=f32)
            y1 = t if y1 is None else y1 + t
    scale1 = big_ref[_R_G1:_R_G1 + 1, :] * _BN_S
    cb1 = big_ref[_R_B1C:_R_B1C + 1, :] * scale1 + big_ref[_R_B1B:_R_B1B + 1, :]
    y1 = jnp.maximum(y1 * scale1 + cb1, 0.0)

    win = {0: (0,), 1: (1, 2), 2: (3, 4)}
    p = []
    for i2 in range(3):
        for j2 in range(3):
            m = None
            for i in win[i2]:
                for j in win[j2]:
                    b = y1[(5 * i + j) * _N:(5 * i + j + 1) * _N, :]
                    m = b if m is None else jnp.maximum(m, b)
            p.append(m)

    w2t = big_ref[_R_W2:_R_W2 + 9, :]
    q = [jax.lax.dot_general(pb, w2t, (((1,), (1,)), ((), ())),
                             preferred_element_type=f32) for pb in p]
    zt = []
    for i2 in range(3):
        for j2 in range(3):
            acc = None
            for di in range(3):
                for dj in range(3):
                    ti, tj = i2 + di - 1, j2 + dj - 1
                    if 0 <= ti < 3 and 0 <= tj < 3:
                        o = di * 3 + dj
                        c = q[ti * 3 + tj][:, o:o + 1]
                        acc = c if acc is None else acc + c
            zt.append(jnp.maximum(acc * sc2 + cb2, 0.0))

    v0 = zt[0]
    v1 = jnp.maximum(zt[1], zt[2])
    v2 = jnp.maximum(zt[3], zt[6])
    v3 = jnp.maximum(jnp.maximum(zt[4], zt[5]), jnp.maximum(zt[7], zt[8]))
    V = jnp.concatenate([v0, v1, v2, v3], axis=1)
    fc3w = big_ref[_R_FC3W:_R_FC3W + 8, 0:4]
    fc3b = big_ref[_R_FC3B:_R_FC3B + 1, 0:8]
    fc4w = big_ref[_R_FC4W:_R_FC4W + 1, 0:8]
    h3 = jnp.maximum(
        jax.lax.dot_general(V, fc3w, (((1,), (1,)), ((), ())),
                            preferred_element_type=f32) + fc3b, 0.0)
    sigma = jnp.sum(h3 * fc4w, axis=1, keepdims=True)
    rec = 1.0 / (sigma + cb4)

    feats = feat_ref[...]
    f = feats * rec
    G = jax.lax.dot_general(f, f, (((1,), (1,)), ((), ())),
                            preferred_element_type=f32)
    ri = jax.lax.broadcasted_iota(jnp.int32, (_N, _N), 0)
    ci = jax.lax.broadcasted_iota(jnp.int32, (_N, _N), 1)
    eye = (ri == ci).astype(f32)
    gd = G * eye
    nv_col = jnp.sum(gd, axis=1, keepdims=True)
    nv_row = jnp.sum(gd, axis=0, keepdims=True)
    t2 = jnp.maximum(nv_col + nv_row - 2.0 * G, 0.0)
    ae = jnp.exp(-t2)

    def drop_max(_, w_):
        m = jnp.max(w_, axis=1, keepdims=True)
        return jnp.where(w_ == m, -1.0, w_)

    wf = jax.lax.fori_loop(0, _K - 1, drop_max, ae)
    thr = jnp.max(wf, axis=1, keepdims=True)
    ae_m = jnp.where((ae >= thr) & (ri != ci), ae, 0.0)

    A = eye + ae_m
    d = jnp.sum(A, axis=1, keepdims=True) + 1.0
    rc = 1.0 / jnp.sqrt(d)
    An = jax.lax.dot_general(A * rc, eye * rc, (((1,), (0,)), ((), ())),
                             preferred_element_type=f32)
    An2 = jax.lax.dot_general(An, An, (((1,), (0,)), ((), ())),
                              preferred_element_type=f32)
    M = a0 * eye + a1 * An + a2 * An2

    support = jax.lax.dot_general(feats, gcnw_ref[...], (((1,), (0,)), ((), ())),
                                  preferred_element_type=f32)
    outv = jax.lax.dot_general(M, support, (((1,), (0,)), ((), ())),
                               preferred_element_type=f32)
    sg = p2_ref[0:1, :]
    cbg = p2_ref[1:2, :]
    out_ref[...] = jnp.maximum(outv * sg + cbg, 0.0)


@jax.jit
def kernel(features, conv1_w, conv1_b, bn_c1_g, bn_c1_b, conv2_w, conv2_b,
           bn_c2_g, bn_c2_b, fc3_w, fc3_b, fc4_w, fc4_b, gcn_w, gcn_b,
           bn2_g, bn2_b, aifa1, aifa2, aifa3):
    no = gcn_w.shape[1]
    xs = features.reshape(_N, _C, _S).transpose(2, 0, 1).reshape(_S * _N, _C)
    w1m = conv1_w.transpose(2, 3, 1, 0).reshape(9 * _C, _C)
    w2t = conv2_w.reshape(_C, 9).T
    big = jnp.concatenate([
        xs, w1m, w2t,
        jnp.zeros((7, _C), jnp.float32),
        jnp.pad(fc3_w, ((0, 0), (0, _C - 4))),
        bn_c1_g.reshape(1, _C), conv1_b.reshape(1, _C), bn_c1_b.reshape(1, _C),
        jnp.pad(fc3_b.reshape(1, 8), ((0, 0), (0, _C - 8))),
        jnp.pad(fc4_w, ((0, 0), (0, _C - 8))),
        jnp.zeros((3, _C), jnp.float32),
    ], axis=0)

    sgv = bn2_g * _BN_S
    p2 = jnp.stack([sgv, gcn_b * sgv + bn2_b])

    sc2 = bn_c2_g * _BN_S
    aifa = jax.nn.softmax(jnp.concatenate([aifa1, aifa2, aifa3]))
    scal = jnp.concatenate([
        sc2, conv2_b * sc2 + bn_c2_b, fc4_b + _EPS_DIV, aifa,
        jnp.zeros((2,), jnp.float32),
    ])

    return pl.pallas_call(
        _body,
        out_shape=jax.ShapeDtypeStruct((_N, no), jnp.float32),
        in_specs=[pl.BlockSpec(memory_space=pltpu.VMEM)] * 4 +
                 [pl.BlockSpec(memory_space=pltpu.SMEM)],
    )(big, features, gcn_w, p2, scal)


# speedup vs baseline: 1.1574x; 1.1574x over previous
import jax
import jax.numpy as jnp
from jax.experimental import pallas as pl
from jax.experimental.pallas import tpu as pltpu

_N = 128
_C = 64
_S = 25
_K = 26
_D = 1600
_EPS_DIV = 2.220446049250313e-16
_BN_S = 1.0 / (1.0 + 1e-5) ** 0.5

_R_XW = 0
_R_W1 = 128
_R_W2 = 192
_R_FC3W = 256
_R_G1 = 264
_R_B1C = 265
_R_B1B = 266
_R_FC3B = 267
_R_FC4W = 268
_R_SG = 269
_R_CBG = 270
_R_FT = 272
_ROWS = 400


def _body(big_ref, gcnw_ref, scal_ref, out_ref):
    f32 = jnp.float32
    sc2 = scal_ref[0]
    cb2 = scal_ref[1]
    cb4 = scal_ref[2]
    a0 = scal_ref[3]
    a1 = scal_ref[4]
    a2 = scal_ref[5]

    w1a = big_ref[_R_W1:_R_W1 + _C, 0:9 * _C]
    rr = jax.lax.broadcasted_iota(jnp.int32, (9 * _C, 9 * _C), 0)
    mm = jax.lax.broadcasted_iota(jnp.int32, (9 * _C, 9 * _C), 1)
    d2 = (mm == (rr % _C) * 9 + rr // _C).astype(f32)
    w1m = jax.lax.dot_general(d2, w1a, (((1,), (1,)), ((), ())),
                              preferred_element_type=f32)

    xb = [big_ref[_R_XW:_R_XW + _N, q * _C:(q + 1) * _C] for q in range(_S)]
    scale1 = big_ref[_R_G1:_R_G1 + 1, 0:_C] * _BN_S
    cb1 = (big_ref[_R_B1C:_R_B1C + 1, 0:_C] * scale1
           + big_ref[_R_B1B:_R_B1B + 1, 0:_C])
    y = []
    for i in range(5):
        for j in range(5):
            acc = None
            for di in range(3):
                for dj in range(3):
                    qi, qj = i + di - 1, j + dj - 1
                    if 0 <= qi < 5 and 0 <= qj < 5:
                        off = di * 3 + dj
                        w = w1m[off * _C:(off + 1) * _C, :]
                        t = jax.lax.dot_general(
                            xb[qi * 5 + qj], w, (((1,), (0,)), ((), ())),
                            preferred_element_type=f32)
                        acc = t if acc is None else acc + t
            y.append(jnp.maximum(acc * scale1 + cb1, 0.0))

    win = {0: (0,), 1: (1, 2), 2: (3, 4)}
    p = []
    for i2 in range(3):
        for j2 in range(3):
            m = None
            for i in win[i2]:
                for j in win[j2]:
                    b = y[5 * i + j]
                    m = b if m is None else jnp.maximum(m, b)
            p.append(m)

    w2r = big_ref[_R_W2:_R_W2 + _C, 0:9]
    q9 = [jax.lax.dot_general(pb, w2r, (((1,), (0,)), ((), ())),
                              preferred_element_type=f32) for pb in p]
    zt = []
    for i2 in range(3):
        for j2 in range(3):
            acc = None
            for di in range(3):
                for dj in range(3):
                    ti, tj = i2 + di - 1, j2 + dj - 1
                    if 0 <= ti < 3 and 0 <= tj < 3:
                        o = di * 3 + dj
                        c = q9[ti * 3 + tj][:, o:o + 1]
                        acc = c if acc is None else acc + c
            zt.append(jnp.maximum(acc * sc2 + cb2, 0.0))

    v0 = zt[0]
    v1 = jnp.maximum(zt[1], zt[2])
    v2 = jnp.maximum(zt[3], zt[6])
    v3 = jnp.maximum(jnp.maximum(zt[4], zt[5]), jnp.maximum(zt[7], zt[8]))
    V = jnp.concatenate([v0, v1, v2, v3], axis=1)
    fc3w = big_ref[_R_FC3W:_R_FC3W + 8, 0:4]
    fc3b = big_ref[_R_FC3B:_R_FC3B + 1, 0:8]
    fc4w = big_ref[_R_FC4W:_R_FC4W + 1, 0:8]
    h3 = jnp.maximum(
        jax.lax.dot_general(V, fc3w, (((1,), (1,)), ((), ())),
                            preferred_element_type=f32) + fc3b, 0.0)
    sigma = jnp.sum(h3 * fc4w, axis=1, keepdims=True)
    rec = 1.0 / (sigma + cb4)

    feats = big_ref[_R_FT:_R_FT + _N, :]
    f = feats * rec
    G = jax.lax.dot_general(f, f, (((1,), (1,)), ((), ())),
                            preferred_element_type=f32)
    ri = jax.lax.broadcasted_iota(jnp.int32, (_N, _N), 0)
    ci = jax.lax.broadcasted_iota(jnp.int32, (_N, _N), 1)
    eye = (ri == ci).astype(f32)
    gd = G * eye
    nv_col = jnp.sum(gd, axis=1, keepdims=True)
    nv_row = jnp.sum(gd, axis=0, keepdims=True)
    t2 = jnp.maximum(nv_col + nv_row - 2.0 * G, 0.0)
    ae = jnp.exp(-t2)

    def drop_max(_, w_):
        m = jnp.max(w_, axis=1, keepdims=True)
        return jnp.where(w_ == m, -1.0, w_)

    wf = jax.lax.fori_loop(0, _K - 1, drop_max, ae)
    thr = jnp.max(wf, axis=1, keepdims=True)
    ae_m = jnp.where((ae >= thr) & (ri != ci), ae, 0.0)

    A = eye + ae_m
    d = jnp.sum(A, axis=1, keepdims=True) + 1.0
    rc = 1.0 / jnp.sqrt(d)
    An = jax.lax.dot_general(A * rc, eye * rc, (((1,), (0,)), ((), ())),
                             preferred_element_type=f32)
    An2 = jax.lax.dot_general(An, An, (((1,), (0,)), ((), ())),
                              preferred_element_type=f32)
    M = a0 * eye + a1 * An + a2 * An2

    support = jax.lax.dot_general(feats, gcnw_ref[...], (((1,), (0,)), ((), ())),
                                  preferred_element_type=f32)
    outv = jax.lax.dot_general(M, support, (((1,), (0,)), ((), ())),
                               preferred_element_type=f32)
    no = outv.shape[1]
    sg = big_ref[_R_SG:_R_SG + 1, 0:no]
    cbg = big_ref[_R_CBG:_R_CBG + 1, 0:no]
    out_ref[...] = jnp.maximum(outv * sg + cbg, 0.0)


def _padrow(x, width=_D):
    x = x.reshape(1, -1)
    return jnp.pad(x, ((0, 0), (0, width - x.shape[1])))


@jax.jit
def kernel(features, conv1_w, conv1_b, bn_c1_g, bn_c1_b, conv2_w, conv2_b,
           bn_c2_g, bn_c2_b, fc3_w, fc3_b, fc4_w, fc4_b, gcn_w, gcn_b,
           bn2_g, bn2_b, aifa1, aifa2, aifa3):
    no = gcn_w.shape[1]
    xw = features.reshape(_N, _C, _S).transpose(0, 2, 1).reshape(_N, _D)
    sgv = bn2_g * _BN_S
    big = jnp.concatenate([
        xw,
        jnp.pad(conv1_w.reshape(_C, 9 * _C), ((0, 0), (0, _D - 9 * _C))),
        jnp.pad(conv2_w.reshape(_C, 9), ((0, 0), (0, _D - 9))),
        jnp.pad(fc3_w, ((0, 0), (0, _D - 4))),
        _padrow(bn_c1_g), _padrow(conv1_b), _padrow(bn_c1_b),
        _padrow(fc3_b), _padrow(fc4_w),
        _padrow(sgv), _padrow(gcn_b * sgv + bn2_b),
        jnp.zeros((1, _D), jnp.float32),
        features,
    ], axis=0)

    sc2 = bn_c2_g * _BN_S
    aifa = jax.nn.softmax(jnp.concatenate([aifa1, aifa2, aifa3]))
    scal = jnp.concatenate([
        sc2, conv2_b * sc2 + bn_c2_b, fc4_b + _EPS_DIV, aifa,
        jnp.zeros((2,), jnp.float32),
    ])

    return pl.pallas_call(
        _body,
        out_shape=jax.ShapeDtypeStruct((_N, no), jnp.float32),
        in_specs=[pl.BlockSpec(memory_space=pltpu.VMEM)] * 2 +
                 [pl.BlockSpec(memory_space=pltpu.SMEM)],
    )(big, gcn_w, scal)


# speedup vs baseline: 1.2614x; 1.0898x over previous
import jax
import jax.numpy as jnp
from jax.experimental import pallas as pl
from jax.experimental.pallas import tpu as pltpu

_N = 128
_C = 64
_S = 25
_K = 26
_D = 1600
_EPS_DIV = 2.220446049250313e-16
_BN_S = 1.0 / (1.0 + 1e-5) ** 0.5

_R_XW = 0
_R_SM = 128
_R_FT = 131
_ROWS = 259


def _body(big_ref, gcnw_ref, w1a_ref, w2r_ref, fc3w_ref, scal_ref, out_ref):
    f32 = jnp.float32
    sc2 = scal_ref[0]
    cb2 = scal_ref[1]
    cb4 = scal_ref[2]
    a0 = scal_ref[3]
    a1 = scal_ref[4]
    a2 = scal_ref[5]

    w1a = w1a_ref[...]
    rr = jax.lax.broadcasted_iota(jnp.int32, (9 * _C, 9 * _C), 0)
    mm = jax.lax.broadcasted_iota(jnp.int32, (9 * _C, 9 * _C), 1)
    d2 = (mm == (rr % _C) * 9 + rr // _C).astype(f32)
    w1m = jax.lax.dot_general(d2, w1a, (((1,), (1,)), ((), ())),
                              preferred_element_type=f32)

    xb = [big_ref[_R_XW:_R_XW + _N, q * _C:(q + 1) * _C] for q in range(_S)]
    scale1 = big_ref[_R_SM:_R_SM + 1, 0:_C] * _BN_S
    cb1 = (big_ref[_R_SM:_R_SM + 1, _C:2 * _C] * scale1
           + big_ref[_R_SM:_R_SM + 1, 2 * _C:3 * _C])
    y = []
    for i in range(5):
        for j in range(5):
            acc = None
            for di in range(3):
                for dj in range(3):
                    qi, qj = i + di - 1, j + dj - 1
                    if 0 <= qi < 5 and 0 <= qj < 5:
                        off = di * 3 + dj
                        w = w1m[off * _C:(off + 1) * _C, :]
                        t = jax.lax.dot_general(
                            xb[qi * 5 + qj], w, (((1,), (0,)), ((), ())),
                            preferred_element_type=f32)
                        acc = t if acc is None else acc + t
            y.append(jnp.maximum(acc * scale1 + cb1, 0.0))

    win = {0: (0,), 1: (1, 2), 2: (3, 4)}
    p = []
    for i2 in range(3):
        for j2 in range(3):
            m = None
            for i in win[i2]:
                for j in win[j2]:
                    b = y[5 * i + j]
                    m = b if m is None else jnp.maximum(m, b)
            p.append(m)

    w2r = w2r_ref[...]
    q9 = [jax.lax.dot_general(pb, w2r, (((1,), (0,)), ((), ())),
                              preferred_element_type=f32) for pb in p]
    zt = []
    for i2 in range(3):
        for j2 in range(3):
            acc = None
            for di in range(3):
                for dj in range(3):
                    ti, tj = i2 + di - 1, j2 + dj - 1
                    if 0 <= ti < 3 and 0 <= tj < 3:
                        o = di * 3 + dj
                        c = q9[ti * 3 + tj][:, o:o + 1]
                        acc = c if acc is None else acc + c
            zt.append(jnp.maximum(acc * sc2 + cb2, 0.0))

    v0 = zt[0]
    v1 = jnp.maximum(zt[1], zt[2])
    v2 = jnp.maximum(zt[3], zt[6])
    v3 = jnp.maximum(jnp.maximum(zt[4], zt[5]), jnp.maximum(zt[7], zt[8]))
    V = jnp.concatenate([v0, v1, v2, v3], axis=1)
    fc3w = fc3w_ref[...]
    fc3b = big_ref[_R_SM:_R_SM + 1, 192:200]
    fc4w = big_ref[_R_SM:_R_SM + 1, 200:208]
    h3 = jnp.maximum(
        jax.lax.dot_general(V, fc3w, (((1,), (1,)), ((), ())),
                            preferred_element_type=f32) + fc3b, 0.0)
    sigma = jnp.sum(h3 * fc4w, axis=1, keepdims=True)
    rec = 1.0 / (sigma + cb4)

    feats = big_ref[_R_FT:_R_FT + _N, :]
    f = feats * rec
    G = jax.lax.dot_general(f, f, (((1,), (1,)), ((), ())),
                            preferred_element_type=f32)
    ri = jax.lax.broadcasted_iota(jnp.int32, (_N, _N), 0)
    ci = jax.lax.broadcasted_iota(jnp.int32, (_N, _N), 1)
    eye = (ri == ci).astype(f32)
    gd = G * eye
    nv_col = jnp.sum(gd, axis=1, keepdims=True)
    nv_row = jnp.sum(gd, axis=0, keepdims=True)
    t2 = jnp.maximum(nv_col + nv_row - 2.0 * G, 0.0)
    ae = jnp.exp(-t2)

    def drop_max(_, w_):
        m = jnp.max(w_, axis=1, keepdims=True)
        return jnp.where(w_ == m, -1.0, w_)

    wf = jax.lax.fori_loop(0, _K - 1, drop_max, ae)
    thr = jnp.max(wf, axis=1, keepdims=True)
    ae_m = jnp.where((ae >= thr) & (ri != ci), ae, 0.0)

    A = eye + ae_m
    d = jnp.sum(A, axis=1, keepdims=True) + 1.0
    rc = 1.0 / jnp.sqrt(d)
    An = jax.lax.dot_general(A * rc, eye * rc, (((1,), (0,)), ((), ())),
                             preferred_element_type=f32)
    An2 = jax.lax.dot_general(An, An, (((1,), (0,)), ((), ())),
                              preferred_element_type=f32)
    M = a0 * eye + a1 * An + a2 * An2

    support = jax.lax.dot_general(feats, gcnw_ref[...], (((1,), (0,)), ((), ())),
                                  preferred_element_type=f32)
    outv = jax.lax.dot_general(M, support, (((1,), (0,)), ((), ())),
                               preferred_element_type=f32)
    no = outv.shape[1]
    sg = big_ref[_R_SM + 1:_R_SM + 2, 0:no]
    cbg = big_ref[_R_SM + 2:_R_SM + 3, 0:no]
    out_ref[...] = jnp.maximum(outv * sg + cbg, 0.0)


def _padrow(x, width=_D):
    x = x.reshape(1, -1)
    return jnp.pad(x, ((0, 0), (0, width - x.shape[1])))


@jax.jit
def kernel(features, conv1_w, conv1_b, bn_c1_g, bn_c1_b, conv2_w, conv2_b,
           bn_c2_g, bn_c2_b, fc3_w, fc3_b, fc4_w, fc4_b, gcn_w, gcn_b,
           bn2_g, bn2_b, aifa1, aifa2, aifa3):
    no = gcn_w.shape[1]
    xw = features.reshape(_N, _C, _S).transpose(0, 2, 1).reshape(_N, _D)
    sgv = bn2_g * _BN_S
    small = jnp.concatenate([
        bn_c1_g, conv1_b, bn_c1_b, fc3_b, fc4_w[0],
        jnp.zeros((_D - 208,), jnp.float32),
        sgv, jnp.zeros((_D - no,), jnp.float32),
        gcn_b * sgv + bn2_b, jnp.zeros((_D - no,), jnp.float32),
    ]).reshape(3, _D)
    big = jnp.concatenate([xw, small, features], axis=0)

    sc2 = bn_c2_g * _BN_S
    aifa = jax.nn.softmax(jnp.concatenate([aifa1, aifa2, aifa3]))
    scal = jnp.concatenate([
        sc2, conv2_b * sc2 + bn_c2_b, fc4_b + _EPS_DIV, aifa,
        jnp.zeros((2,), jnp.float32),
    ])

    return pl.pallas_call(
        _body,
        out_shape=jax.ShapeDtypeStruct((_N, no), jnp.float32),
        in_specs=[pl.BlockSpec(memory_space=pltpu.VMEM)] * 5 +
                 [pl.BlockSpec(memory_space=pltpu.SMEM)],
    )(big, gcn_w, conv1_w.reshape(_C, 9 * _C), conv2_w.reshape(_C, 9),
      fc3_w, scal)


# speedup vs baseline: 1.2719x; 1.0084x over previous
import jax
import jax.numpy as jnp
from jax.experimental import pallas as pl
from jax.experimental.pallas import tpu as pltpu

_N = 128
_C = 64
_S = 25
_K = 26
_D = 1600
_EPS_DIV = 2.220446049250313e-16
_BN_S = 1.0 / (1.0 + 1e-5) ** 0.5

_R_XW = 0
_R_SM = 128
_R_FT = 131
_ROWS = 259


def _body(big_ref, gcnw_ref, w1a_ref, w2r_ref, fc3w_ref, scal_ref, out_ref):
    f32 = jnp.float32
    sc2 = scal_ref[0]
    cb2 = scal_ref[1]
    cb4 = scal_ref[2]
    a0 = scal_ref[3]
    a1 = scal_ref[4]
    a2 = scal_ref[5]

    w1a = w1a_ref[...]
    rr = jax.lax.broadcasted_iota(jnp.int32, (9 * _C, 9 * _C), 0)
    mm = jax.lax.broadcasted_iota(jnp.int32, (9 * _C, 9 * _C), 1)
    d2 = (mm == (rr % _C) * 9 + rr // _C).astype(f32)
    w1m = jax.lax.dot_general(d2, w1a, (((1,), (1,)), ((), ())),
                              preferred_element_type=f32)

    xb = [big_ref[_R_XW:_R_XW + _N, q * _C:(q + 1) * _C] for q in range(_S)]
    scale1 = big_ref[_R_SM:_R_SM + 1, 0:_C] * _BN_S
    cb1 = (big_ref[_R_SM:_R_SM + 1, _C:2 * _C] * scale1
           + big_ref[_R_SM:_R_SM + 1, 2 * _C:3 * _C])
    y = []
    for i in range(5):
        for j in range(5):
            acc = None
            for di in range(3):
                for dj in range(3):
                    qi, qj = i + di - 1, j + dj - 1
                    if 0 <= qi < 5 and 0 <= qj < 5:
                        off = di * 3 + dj
                        w = w1m[off * _C:(off + 1) * _C, :]
                        t = jax.lax.dot_general(
                            xb[qi * 5 + qj], w, (((1,), (0,)), ((), ())),
                            preferred_element_type=f32)
                        acc = t if acc is None else acc + t
            y.append(jnp.maximum(acc * scale1 + cb1, 0.0))

    win = {0: (0,), 1: (1, 2), 2: (3, 4)}
    p = []
    for i2 in range(3):
        for j2 in range(3):
            m = None
            for i in win[i2]:
                for j in win[j2]:
                    b = y[5 * i + j]
                    m = b if m is None else jnp.maximum(m, b)
            p.append(m)

    w2r = w2r_ref[...]
    q9 = [jax.lax.dot_general(pb, w2r, (((1,), (0,)), ((), ())),
                              preferred_element_type=f32) for pb in p]
    zt = []
    for i2 in range(3):
        for j2 in range(3):
            acc = None
            for di in range(3):
                for dj in range(3):
                    ti, tj = i2 + di - 1, j2 + dj - 1
                    if 0 <= ti < 3 and 0 <= tj < 3:
                        o = di * 3 + dj
                        c = q9[ti * 3 + tj][:, o:o + 1]
                        acc = c if acc is None else acc + c
            zt.append(jnp.maximum(acc * sc2 + cb2, 0.0))

    v0 = zt[0]
    v1 = jnp.maximum(zt[1], zt[2])
    v2 = jnp.maximum(zt[3], zt[6])
    v3 = jnp.maximum(jnp.maximum(zt[4], zt[5]), jnp.maximum(zt[7], zt[8]))
    V = jnp.concatenate([v0, v1, v2, v3], axis=1)
    fc3w = fc3w_ref[...]
    fc3b = big_ref[_R_SM:_R_SM + 1, 192:200]
    fc4w = big_ref[_R_SM:_R_SM + 1, 200:208]
    h3 = jnp.maximum(
        jax.lax.dot_general(V, fc3w, (((1,), (1,)), ((), ())),
                            preferred_element_type=f32) + fc3b, 0.0)
    sigma = jnp.sum(h3 * fc4w, axis=1, keepdims=True)
    rec = 1.0 / (sigma + cb4)

    feats = big_ref[_R_FT:_R_FT + _N, :]
    f = feats * rec
    G = jax.lax.dot_general(f, f, (((1,), (1,)), ((), ())),
                            preferred_element_type=f32)
    ri = jax.lax.broadcasted_iota(jnp.int32, (_N, _N), 0)
    ci = jax.lax.broadcasted_iota(jnp.int32, (_N, _N), 1)
    eye = (ri == ci).astype(f32)
    gd = G * eye
    nv_col = jnp.sum(gd, axis=1, keepdims=True)
    nv_row = jnp.sum(gd, axis=0, keepdims=True)
    t2 = jnp.maximum(nv_col + nv_row - 2.0 * G, 0.0)
    ae = jnp.exp(-t2)

    def drop_max(_, w_):
        m = jnp.max(w_, axis=1, keepdims=True)
        return jnp.where(w_ == m, -1.0, w_)

    wf = jax.lax.fori_loop(0, _K - 1, drop_max, ae)
    thr = jnp.max(wf, axis=1, keepdims=True)
    ae_m = jnp.where((ae >= thr) & (ri != ci), ae, 0.0)

    A = eye + ae_m
    d = jnp.sum(A, axis=1, keepdims=True) + 1.0
    rc = 1.0 / jnp.sqrt(d)
    An = jax.lax.dot_general(A * rc, eye * rc, (((1,), (0,)), ((), ())),
                             preferred_element_type=f32)
    An2 = jax.lax.dot_general(An, An, (((1,), (0,)), ((), ())),
                              preferred_element_type=f32)
    M = a0 * eye + a1 * An + a2 * An2

    support = jax.lax.dot_general(feats, gcnw_ref[...], (((1,), (0,)), ((), ())),
                                  preferred_element_type=f32)
    outv = jax.lax.dot_general(M, support, (((1,), (0,)), ((), ())),
                               preferred_element_type=f32)
    no = outv.shape[1]
    sg = big_ref[_R_SM + 1:_R_SM + 2, 0:no]
    cbg = big_ref[_R_SM + 2:_R_SM + 3, 0:no]
    out_ref[...] = jnp.maximum(outv * sg + cbg, 0.0)


@jax.jit
def kernel(features, conv1_w, conv1_b, bn_c1_g, bn_c1_b, conv2_w, conv2_b,
           bn_c2_g, bn_c2_b, fc3_w, fc3_b, fc4_w, fc4_b, gcn_w, gcn_b,
           bn2_g, bn2_b, aifa1, aifa2, aifa3):
    no = gcn_w.shape[1]
    xw = features.reshape(_N, _C, _S).transpose(0, 2, 1).reshape(_N, _D)
    sgv = bn2_g * _BN_S
    small = jnp.concatenate([
        bn_c1_g, conv1_b, bn_c1_b, fc3_b, fc4_w[0],
        jnp.zeros((_D - 208,), jnp.float32),
        sgv, jnp.zeros((_D - no,), jnp.float32),
        gcn_b * sgv + bn2_b, jnp.zeros((_D - no,), jnp.float32),
    ]).reshape(3, _D)
    big = jnp.concatenate([xw, small, features], axis=0)

    sc2 = bn_c2_g * _BN_S
    aifa = jax.nn.softmax(jnp.concatenate([aifa1, aifa2, aifa3]))
    scal = jnp.concatenate([
        sc2, conv2_b * sc2 + bn_c2_b, fc4_b + _EPS_DIV, aifa,
        jnp.zeros((2,), jnp.float32),
    ])

    return pl.pallas_call(
        _body,
        out_shape=jax.ShapeDtypeStruct((_N, no), jnp.float32),
        in_specs=[pl.BlockSpec(memory_space=pltpu.VMEM)] * 5 +
                 [pl.BlockSpec(memory_space=pltpu.SMEM)],
    )(big, gcn_w, conv1_w.reshape(_C, 9 * _C), conv2_w.reshape(_C, 9),
      fc3_w, scal)
